# baseline (speedup 1.0000x reference)
import jax
import jax.numpy as jnp
from jax.experimental import pallas as pl
from jax.experimental.pallas import tpu as pltpu

_WINDOW = 256
_TAU_INTER = 0.5
_TAU_TEMP = 1.2
_MAX_AGE = 4


def _body(b, d, x_ref, scale_ref, bias_ref, o_ref, sig_ref, age_ref):
    w = pl.program_id(0)
    xw = x_ref[0]
    tw = xw.shape[0]

    sig = jnp.mean(xw, axis=0, keepdims=True)
    dsum = jnp.sum(jnp.abs(xw[1:] - xw[:-1]), axis=0, keepdims=True)
    prev = sig_ref[...]
    delta = sig - prev
    dd = delta * delta

    sc = scale_ref[...]
    bi = bias_ref[...]

    for bb in range(b):
        lo = bb * d
        d2b = jnp.sum(dd[0, lo : lo + d])
        vtb = jnp.sum(dsum[0, lo : lo + d]) * (1.0 / ((tw - 1) * d))
        ageb = age_ref[bb]
        refresh = (
            (w == 0)
            | (ageb >= _MAX_AGE)
            | (d2b > _TAU_INTER * _TAU_INTER * d)
            | (vtb > _TAU_TEMP)
        )

        def _do_refresh(lo=lo, bb=bb):
            o_ref[0, :, lo : lo + d] = xw[:, lo : lo + d] * sc + bi
            sig_ref[0, lo : lo + d] = sig[0, lo : lo + d]
            age_ref[bb] = 0

        def _do_reuse(lo=lo, bb=bb, ageb=ageb):
            row = prev[:, lo : lo + d] * sc + bi
            o_ref[0, :, lo : lo + d] = jnp.broadcast_to(row, (tw, d))
            age_ref[bb] = ageb + 1

        pl.when(refresh)(_do_refresh)
        pl.when(jnp.logical_not(refresh))(_do_reuse)


def kernel(x, scale, bias):
    t, b, d = x.shape
    nw = t // _WINDOW
    xr = x.reshape(nw, _WINDOW, b * d)
    body = lambda *refs: _body(b, d, *refs)
    out = pl.pallas_call(
        body,
        grid=(nw,),
        in_specs=[
            pl.BlockSpec((1, _WINDOW, b * d), lambda w: (w, 0, 0)),
            pl.BlockSpec((1, d), lambda w: (0, 0)),
            pl.BlockSpec((1, d), lambda w: (0, 0)),
        ],
        out_specs=pl.BlockSpec((1, _WINDOW, b * d), lambda w: (w, 0, 0)),
        out_shape=jax.ShapeDtypeStruct((nw, _WINDOW, b * d), x.dtype),
        scratch_shapes=[
            pltpu.VMEM((1, b * d), jnp.float32),
            pltpu.SMEM((b,), jnp.int32),
        ],
    )(xr, scale.reshape(1, d), bias.reshape(1, d))
    return out.reshape(t, b, d)


# speedup vs baseline: 1.0188x; 1.0188x over previous
import jax
import jax.numpy as jnp
from jax.experimental import pallas as pl
from jax.experimental.pallas import tpu as pltpu

_WINDOW = 256
_TAU_INTER = 0.5
_TAU_TEMP = 1.2
_MAX_AGE = 4


def _body(b, d, x_ref, scale_ref, bias_ref, o_ref, sig_ref, age_ref):
    w = pl.program_id(0)
    xw = x_ref[0]
    tw = xw.shape[0]

    sig = jnp.mean(xw, axis=0, keepdims=True)
    dsum = jnp.sum(jnp.abs(xw[1:] - xw[:-1]), axis=0, keepdims=True)
    prev = sig_ref[...]
    delta = sig - prev
    dd = delta * delta

    mask_parts = []
    for bb in range(b):
        lo = bb * d
        d2b = jnp.sum(dd[0, lo : lo + d])
        vtb = jnp.sum(dsum[0, lo : lo + d]) * (1.0 / ((tw - 1) * d))
        ageb = age_ref[bb]
        refresh = (
            (w == 0)
            | (ageb >= _MAX_AGE)
            | (d2b > _TAU_INTER * _TAU_INTER * d)
            | (vtb > _TAU_TEMP)
        )
        age_ref[bb] = jnp.where(refresh, 0, ageb + 1)
        mask_parts.append(jnp.broadcast_to(refresh, (1, d)))
    mask = jnp.concatenate(mask_parts, axis=1)

    new_sig = jnp.where(mask, sig, prev)
    sig_ref[...] = new_sig

    sc = scale_ref[...]
    bi = bias_ref[...]
    y_full = xw * sc + bi
    y_reuse = new_sig * sc + bi
    o_ref[0] = jnp.where(mask, y_full, y_reuse)


def kernel(x, scale, bias):
    t, b, d = x.shape
    nw = t // _WINDOW
    xr = x.reshape(nw, _WINDOW, b * d)
    scale_t = jnp.tile(scale, (b,)).reshape(1, b * d)
    bias_t = jnp.tile(bias, (b,)).reshape(1, b * d)
    body = lambda *refs: _body(b, d, *refs)
    out = pl.pallas_call(
        body,
        grid=(nw,),
        in_specs=[
            pl.BlockSpec((1, _WINDOW, b * d), lambda w: (w, 0, 0)),
            pl.BlockSpec((1, b * d), lambda w: (0, 0)),
            pl.BlockSpec((1, b * d), lambda w: (0, 0)),
        ],
        out_specs=pl.BlockSpec((1, _WINDOW, b * d), lambda w: (w, 0, 0)),
        out_shape=jax.ShapeDtypeStruct((nw, _WINDOW, b * d), x.dtype),
        scratch_shapes=[
            pltpu.VMEM((1, b * d), jnp.float32),
            pltpu.SMEM((b,), jnp.int32),
        ],
    )(xr, scale_t, bias_t)
    return out.reshape(t, b, d)


# speedup vs baseline: 3.2678x; 3.2076x over previous
import jax
import jax.numpy as jnp
from jax.experimental import pallas as pl
from jax.experimental.pallas import tpu as pltpu

_WINDOW = 256
_TAU_INTER = 0.5
_TAU_TEMP = 1.2
_MAX_AGE = 4


def _body(x_ref, scale_ref, bias_ref, o_ref, sig_ref, age_ref):
    w = pl.program_id(0)
    xw = x_ref[...]
    tw, b, d = xw.shape

    sig = jnp.sum(xw, axis=0) * (1.0 / tw)
    ad0 = jnp.sum(jnp.abs(xw[1:] - xw[:-1]), axis=0)
    vt = jnp.sum(ad0, axis=1, keepdims=True) * (1.0 / ((tw - 1) * d))

    prev = sig_ref[...]
    delta = sig - prev
    d2 = jnp.sum(delta * delta, axis=1, keepdims=True)

    age = age_ref[...]
    refresh = (
        (w == 0)
        | (age >= _MAX_AGE)
        | (d2 > _TAU_INTER * _TAU_INTER * d)
        | (vt > _TAU_TEMP)
    )

    new_sig = jnp.where(refresh, sig, prev)
    sig_ref[...] = new_sig
    age_ref[...] = jnp.where(refresh, 0, age + 1)

    sc = scale_ref[...]
    bi = bias_ref[...]
    sc_eff = jnp.where(refresh, sc, 0.0)
    bi_eff = jnp.where(refresh, bi, new_sig * sc + bi)
    o_ref[...] = xw * sc_eff[None] + bi_eff[None]


def kernel(x, scale, bias):
    t, b, d = x.shape
    nw = t // _WINDOW
    out = pl.pallas_call(
        _body,
        grid=(nw,),
        in_specs=[
            pl.BlockSpec((_WINDOW, b, d), lambda w: (w, 0, 0)),
            pl.BlockSpec((1, d), lambda w: (0, 0)),
            pl.BlockSpec((1, d), lambda w: (0, 0)),
        ],
        out_specs=pl.BlockSpec((_WINDOW, b, d), lambda w: (w, 0, 0)),
        out_shape=jax.ShapeDtypeStruct((t, b, d), x.dtype),
        scratch_shapes=[
            pltpu.VMEM((b, d), jnp.float32),
            pltpu.VMEM((b, 1), jnp.int32),
        ],
    )(x, scale.reshape(1, d), bias.reshape(1, d))
    return out


# speedup vs baseline: 3.3085x; 1.0124x over previous
import jax
import jax.numpy as jnp
from jax.experimental import pallas as pl
from jax.experimental.pallas import tpu as pltpu

_WINDOW = 256
_NQ = 4
_Q = _WINDOW // _NQ
_TAU_INTER = 0.5
_TAU_TEMP = 1.2
_MAX_AGE = 4


def _body(x0, x1, x2, x3, scale_ref, bias_ref, o_ref, ybuf, sig_ref,
          age_ref, sem):
    w = pl.program_id(0)
    nw = pl.num_programs(0)
    par = jax.lax.rem(w, 2)

    def _wait(step, parity):
        for q in range(_NQ):
            pltpu.make_async_copy(
                ybuf.at[parity, pl.ds(q * _Q, _Q)],
                o_ref.at[pl.ds(step * _WINDOW + q * _Q, _Q)],
                sem.at[parity],
            ).wait()

    @pl.when(w >= 2)
    def _():
        _wait(w - 2, par)

    xqs = [x0[...], x1[...], x2[...], x3[...]]
    b = xqs[0].shape[1]
    d = xqs[0].shape[2]

    total = jnp.sum(xqs[0], axis=0)
    for xq in xqs[1:]:
        total = total + jnp.sum(xq, axis=0)
    sig = total * (1.0 / _WINDOW)

    ad0 = jnp.sum(jnp.abs(xqs[0][1:] - xqs[0][:-1]), axis=0)
    for xq in xqs[1:]:
        ad0 = ad0 + jnp.sum(jnp.abs(xq[1:] - xq[:-1]), axis=0)
    for qi in range(_NQ - 1):
        ad0 = ad0 + jnp.abs(xqs[qi + 1][0] - xqs[qi][-1])
    vt = jnp.sum(ad0, axis=1, keepdims=True) * (
        1.0 / ((_WINDOW - 1) * d)
    )

    prev = sig_ref[...]
    delta = sig - prev
    d2 = jnp.sum(delta * delta, axis=1, keepdims=True)

    age = age_ref[...]
    refresh = (
        (w == 0)
        | (age >= _MAX_AGE)
        | (d2 > _TAU_INTER * _TAU_INTER * d)
        | (vt > _TAU_TEMP)
    )

    new_sig = jnp.where(refresh, sig, prev)
    sig_ref[...] = new_sig
    age_ref[...] = jnp.where(refresh, 0, age + 1)

    sc = scale_ref[...]
    bi = bias_ref[...]
    sc_eff = jnp.where(refresh, sc, 0.0)
    bi_eff = jnp.where(refresh, bi, new_sig * sc + bi)

    for qi in range(_NQ):
        ybuf[par, pl.ds(qi * _Q, _Q)] = (
            xqs[qi] * sc_eff[None] + bi_eff[None]
        )

    for q in range(_NQ):
        pltpu.make_async_copy(
            ybuf.at[par, pl.ds(q * _Q, _Q)],
            o_ref.at[pl.ds(w * _WINDOW + q * _Q, _Q)],
            sem.at[par],
        ).start()

    @pl.when(w == nw - 1)
    def _():
        _wait(w - 1, 1 - par)
        _wait(w, par)


def kernel(x, scale, bias):
    t, b, d = x.shape
    nw = t // _WINDOW
    in_specs = [
        pl.BlockSpec((_Q, b, d), (lambda w, qi=qi: (w * _NQ + qi, 0, 0)))
        for qi in range(_NQ)
    ] + [
        pl.BlockSpec((1, d), lambda w: (0, 0)),
        pl.BlockSpec((1, d), lambda w: (0, 0)),
    ]
    out = pl.pallas_call(
        _body,
        grid=(nw,),
        in_specs=in_specs,
        out_specs=pl.BlockSpec(memory_space=pltpu.MemorySpace.HBM),
        out_shape=jax.ShapeDtypeStruct((t, b, d), x.dtype),
        scratch_shapes=[
            pltpu.VMEM((2, _WINDOW, b, d), jnp.float32),
            pltpu.VMEM((b, d), jnp.float32),
            pltpu.VMEM((b, 1), jnp.int32),
            pltpu.SemaphoreType.DMA((2,)),
        ],
    )(x, x, x, x, scale.reshape(1, d), bias.reshape(1, d))
    return out


# speedup vs baseline: 3.3164x; 1.0024x over previous
import jax
import jax.numpy as jnp
from jax.experimental import pallas as pl
from jax.experimental.pallas import tpu as pltpu

_WINDOW = 256
_NQ = 4
_Q = _WINDOW // _NQ
_TAU_INTER = 0.5
_TAU_TEMP = 1.2
_MAX_AGE = 4


def _body(x0, x1, x2, x3, scale_ref, bias_ref, o_ref, ybuf, sig_ref,
          age_ref, sem):
    w = pl.program_id(0)
    nw = pl.num_programs(0)
    par = jax.lax.rem(w, 2)

    def _wait(step, parity):
        for q in range(_NQ):
            pltpu.make_async_copy(
                ybuf.at[parity, pl.ds(q * _Q, _Q)],
                o_ref.at[pl.ds(step * _WINDOW + q * _Q, _Q)],
                sem.at[parity],
            ).wait()

    @pl.when(w >= 2)
    def _():
        _wait(w - 2, par)

    xrefs = [x0, x1, x2, x3]
    b = x0.shape[1]
    d = x0.shape[2]
    sc = scale_ref[...]
    bi = bias_ref[...]

    zero2 = jnp.zeros((2, b, d), jnp.float32)
    sum2 = zero2
    ad2 = zero2
    prev = None
    for qi in range(_NQ):
        xr = xrefs[qi]

        def pair_body(i, carry, qi=qi, xr=xr):
            sum2, ad2, prev = carry
            cur2 = xr[pl.ds(2 * i, 2)]
            ybuf[par, pl.ds(qi * _Q + 2 * i, 2)] = cur2 * sc[None] + bi[None]
            shifted = jnp.concatenate([prev, cur2[:1]], axis=0)
            return (
                sum2 + cur2,
                ad2 + jnp.abs(cur2 - shifted),
                cur2[1:2],
            )

        if qi == 0:
            first = x0[pl.ds(0, 2)]
            ybuf[par, pl.ds(0, 2)] = first * sc[None] + bi[None]
            sum2 = sum2 + first
            ad2 = ad2 + jnp.abs(
                first - jnp.concatenate([first[:1], first[:1]], axis=0)
            )
            prev = first[1:2]
            lo = 1
        else:
            lo = 0
        sum2, ad2, prev = jax.lax.fori_loop(
            lo, _Q // 2, pair_body, (sum2, ad2, prev)
        )

    sig = (sum2[0] + sum2[1]) * (1.0 / _WINDOW)
    ad = ad2[0] + ad2[1]

    prev_sig = sig_ref[...]
    delta = sig - prev_sig
    d2 = jnp.sum(delta * delta, axis=1, keepdims=True)
    vt = jnp.sum(ad, axis=1, keepdims=True) * (
        1.0 / ((_WINDOW - 1) * d)
    )

    age = age_ref[...]
    refresh = (
        (w == 0)
        | (age >= _MAX_AGE)
        | (d2 > _TAU_INTER * _TAU_INTER * d)
        | (vt > _TAU_TEMP)
    )

    new_sig = jnp.where(refresh, sig, prev_sig)
    sig_ref[...] = new_sig
    age_ref[...] = jnp.where(refresh, 0, age + 1)

    refresh_i = refresh.astype(jnp.int32)
    for bb in range(b):
        @pl.when(refresh_i[bb, 0] == 0)
        def _(bb=bb):
            row = new_sig[bb : bb + 1] * sc + bi
            ybuf[par, :, bb, :] = jnp.broadcast_to(row, (_WINDOW, d))

    for q in range(_NQ):
        pltpu.make_async_copy(
            ybuf.at[par, pl.ds(q * _Q, _Q)],
            o_ref.at[pl.ds(w * _WINDOW + q * _Q, _Q)],
            sem.at[par],
        ).start()

    @pl.when(w == nw - 1)
    def _():
        _wait(w - 1, 1 - par)
        _wait(w, par)


def kernel(x, scale, bias):
    t, b, d = x.shape
    nw = t // _WINDOW
    in_specs = [
        pl.BlockSpec((_Q, b, d), (lambda w, qi=qi: (w * _NQ + qi, 0, 0)))
        for qi in range(_NQ)
    ] + [pl.BlockSpec((1, d), lambda w: (0, 0))] * 2
    out = pl.pallas_call(
        _body,
        grid=(nw,),
        in_specs=in_specs,
        out_specs=pl.BlockSpec(memory_space=pltpu.MemorySpace.HBM),
        out_shape=jax.ShapeDtypeStruct((t, b, d), x.dtype),
        scratch_shapes=[
            pltpu.VMEM((2, _WINDOW, b, d), jnp.float32),
            pltpu.VMEM((b, d), jnp.float32),
            pltpu.VMEM((b, 1), jnp.int32),
            pltpu.SemaphoreType.DMA((2,)),
        ],
    )(x, x, x, x, scale.reshape(1, d), bias.reshape(1, d))
    return out
